# baseline (speedup 1.0000x reference)
import functools

import jax
import jax.numpy as jnp
from jax import lax
from jax.experimental import pallas as pl
from jax.experimental.pallas import tpu as pltpu
from jax.experimental.pallas import tpu_sc as plsc

N = 10000
NP = 10240
E = 320000
F_IN = 128
H = 128
C = 64

NC = 2
NS = 16
NW = NC * NS
L = 16
B = 80
NBLK = 128
EPW = NBLK * B
EP = NW * EPW
RPT = NP // NS
CHB = 32
CHE = CHB * B
NCH = NBLK // CHB

_MESH = dict(core_axis_name="c", subcore_axis_name="s",
             num_cores=NC, num_subcores=NS)
_SC_PARAMS = pltpu.CompilerParams(needs_layout_passes=False)

_GDN = lax.GatherDimensionNumbers(
    offset_dims=(), collapsed_slice_dims=(0,), start_index_map=(0,))


def _lane_bcast(v16, j):
    return lax.gather(v16, jnp.full((L, 1), j, jnp.int32), _GDN, (1,),
                      mode=lax.GatherScatterMode.PROMISE_IN_BOUNDS)


def _sc_degree(dst2, ew2):
    @functools.partial(
        pl.kernel,
        out_type=jax.ShapeDtypeStruct((NW, NP), jnp.float32),
        mesh=plsc.VectorSubcoreMesh(**_MESH),
        compiler_params=_SC_PARAMS,
        scratch_types=[
            pltpu.VMEM((EPW,), jnp.int32),
            pltpu.VMEM((EPW,), jnp.float32),
            pltpu.VMEM((NP,), jnp.float32),
        ],
    )
    def k(dst_hbm, ew_hbm, out_hbm, dst_v, ew_v, acc):
        cid = lax.axis_index("c")
        sid = lax.axis_index("s")
        wid = sid * NC + cid
        pltpu.sync_copy(dst_hbm.at[wid], dst_v)
        pltpu.sync_copy(ew_hbm.at[wid], ew_v)

        zeros = jnp.zeros((L,), jnp.float32)

        def zbody(i, _):
            acc[pl.ds(i * L, L)] = zeros
            return 0

        lax.fori_loop(0, NP // L, zbody, 0)

        def ebody(i, _):
            idx = dst_v[pl.ds(i * L, L)]
            w = ew_v[pl.ds(i * L, L)]
            plsc.addupdate_scatter(acc, [idx], w)
            return 0

        lax.fori_loop(0, EPW // L, ebody, 0)
        pltpu.sync_copy(acc, out_hbm.at[wid])

    return k(dst2, ew2)


def _make_sc_aggregate(D):
    @functools.partial(
        pl.kernel,
        out_type=jax.ShapeDtypeStruct((NC, NP, D), jnp.float32),
        mesh=plsc.VectorSubcoreMesh(**_MESH),
        compiler_params=_SC_PARAMS,
        scratch_types=[
            pltpu.VMEM((CHE,), jnp.int32),
            pltpu.VMEM((CHE,), jnp.int32),
            pltpu.VMEM((CHE,), jnp.float32),
            pltpu.VMEM((B,), jnp.int32),
            pltpu.VMEM((B,), jnp.int32),
            pltpu.VMEM((B,), jnp.int32),
            pltpu.VMEM((B,), jnp.int32),
            pltpu.VMEM((B,), jnp.float32),
            pltpu.VMEM((B,), jnp.float32),
            pltpu.VMEM((B, D), jnp.float32),
            pltpu.VMEM((B, D), jnp.float32),
            pltpu.VMEM_SHARED((NP, D), jnp.float32),
            pltpu.SemaphoreType.DMA,
            pltpu.SemaphoreType.DMA,
            pltpu.SemaphoreType.DMA,
            pltpu.SemaphoreType.DMA,
        ],
    )
    def k(g_hbm, src_hbm, dst_hbm, ew_hbm, out_hbm,
          src_all, dst_all, ew_all,
          sv0, sv1, dv0, dv1, wv0, wv1, rows0, rows1, acc_sh,
          gs0, gs1, ss0, ss1):
        sv = (sv0, sv1)
        dv = (dv0, dv1)
        wv = (wv0, wv1)
        rows = (rows0, rows1)
        gs = (gs0, gs1)
        ss = (ss0, ss1)
        cid = lax.axis_index("c")
        sid = lax.axis_index("s")
        wid = sid * NC + cid

        zeros = jnp.zeros((L,), jnp.float32)

        def zbody(i, _):
            for kk in range(D // L):
                rows0[i, pl.ds(kk * L, L)] = zeros
            return 0

        lax.fori_loop(0, B, zbody, 0)
        rbase = sid * RPT
        for cchunk in range(RPT // B):
            pltpu.sync_copy(rows0, acc_sh.at[pl.ds(rbase + cchunk * B, B)])
        plsc.subcore_barrier()

        def stage_chunk(ch):
            pltpu.sync_copy(src_hbm.at[wid, ch], src_all)
            pltpu.sync_copy(dst_hbm.at[wid, ch], dst_all)
            pltpu.sync_copy(ew_hbm.at[wid, ch], ew_all)

        def prep(off, p):
            for g in range(B // L):
                sl = pl.ds(off + g * L, L)
                sv[p][pl.ds(g * L, L)] = src_all[sl]
                dv[p][pl.ds(g * L, L)] = dst_all[sl]
                wv[p][pl.ds(g * L, L)] = ew_all[sl]

        def scale(p):
            def sgroup(g, _):
                w16 = wv[p][pl.ds(g * L, L)]
                for j in range(L):
                    e = g * L + j
                    wspl = _lane_bcast(w16, j)
                    r = rows[p]
                    for kk in range(D // L):
                        r[e, pl.ds(kk * L, L)] = r[e, pl.ds(kk * L, L)] * wspl
                return 0

            lax.fori_loop(0, B // L, sgroup, 0)

        stage_chunk(0)
        prep(0, 0)
        pltpu.async_copy(g_hbm.at[sv0], rows0, gs0)

        def pair(i, _):
            for p in range(2):
                b = i * 2 + p
                q = 1 - p
                nxt = b + 1
                pltpu.make_async_copy(g_hbm.at[sv[p]], rows[p],
                                      gs[p]).wait()

                @pl.when((nxt % CHB == 0) & (nxt < NBLK))
                def _():
                    stage_chunk(nxt // CHB)

                @pl.when(nxt < NBLK)
                def _():
                    prep(jnp.mod(nxt, CHB) * B, q)

                scale(p)
                pltpu.async_copy(rows[p], acc_sh.at[dv[p]], ss[p],
                                 add=True)

                @pl.when(nxt < NBLK)
                def _():
                    @pl.when(b >= 1)
                    def _():
                        pltpu.make_async_copy(g_hbm.at[sv[q]], rows[q],
                                              ss[q]).wait()

                    pltpu.async_copy(g_hbm.at[sv[q]], rows[q], gs[q])
            return 0

        lax.fori_loop(0, NBLK // 2, pair, 0)
        pltpu.make_async_copy(g_hbm.at[sv0], rows0, ss0).wait()
        pltpu.make_async_copy(g_hbm.at[sv1], rows1, ss1).wait()
        plsc.subcore_barrier()
        pltpu.sync_copy(acc_sh.at[pl.ds(rbase, RPT)],
                        out_hbm.at[cid, pl.ds(rbase, RPT)])

    return k


_sc_aggregate_h = _make_sc_aggregate(H)


_R = 1024
_G = NP // _R


def _tc_prep(degp, x_pad, W1):
    def body(deg_ref, x_ref, w_ref, g_ref):
        deg = jnp.sum(deg_ref[...], axis=0) + 1.0
        di = lax.rsqrt(deg)[:, None]
        h = jnp.dot(x_ref[...], w_ref[...],
                    preferred_element_type=jnp.float32)
        g_ref[...] = di * h

    return pl.pallas_call(
        body,
        grid=(_G,),
        in_specs=[
            pl.BlockSpec((NW, _R), lambda i: (0, i)),
            pl.BlockSpec((_R, F_IN), lambda i: (i, 0)),
            pl.BlockSpec((F_IN, H), lambda i: (0, 0)),
        ],
        out_specs=pl.BlockSpec((_R, H), lambda i: (i, 0)),
        out_shape=jax.ShapeDtypeStruct((NP, H), jnp.float32),
    )(degp, x_pad, W1)


def _tc_mid(acc1p, degp, g1, b1r):
    def body(acc_ref, deg_ref, g_ref, b_ref, q2_ref):
        deg = jnp.sum(deg_ref[...], axis=0) + 1.0
        di = lax.rsqrt(deg)[:, None]
        a = acc_ref[0] + acc_ref[1] + g_ref[...]
        out1 = jnp.maximum(di * a + b_ref[...], 0.0)
        q2_ref[...] = di * out1

    return pl.pallas_call(
        body,
        grid=(_G,),
        in_specs=[
            pl.BlockSpec((NC, _R, H), lambda i: (0, i, 0)),
            pl.BlockSpec((NW, _R), lambda i: (0, i)),
            pl.BlockSpec((_R, H), lambda i: (i, 0)),
            pl.BlockSpec((1, H), lambda i: (0, 0)),
        ],
        out_specs=pl.BlockSpec((_R, H), lambda i: (i, 0)),
        out_shape=jax.ShapeDtypeStruct((NP, H), jnp.float32),
    )(acc1p, degp, g1, b1r)


def _tc_final(acc2p, degp, q2, W2, b2r):
    def body(acc_ref, deg_ref, q_ref, w_ref, b_ref, o_ref):
        deg = jnp.sum(deg_ref[...], axis=0) + 1.0
        di = lax.rsqrt(deg)[:, None]
        a = di * (acc_ref[0] + acc_ref[1] + q_ref[...])
        o = jnp.dot(a, w_ref[...],
                    preferred_element_type=jnp.float32) + b_ref[...]
        nrm = jnp.sqrt(jnp.sum(o * o, axis=1, keepdims=True))
        o_ref[...] = o / jnp.maximum(nrm, 1e-12)

    return pl.pallas_call(
        body,
        grid=(_G,),
        in_specs=[
            pl.BlockSpec((NC, _R, H), lambda i: (0, i, 0)),
            pl.BlockSpec((NW, _R), lambda i: (0, i)),
            pl.BlockSpec((_R, H), lambda i: (i, 0)),
            pl.BlockSpec((H, C), lambda i: (0, 0)),
            pl.BlockSpec((1, C), lambda i: (0, 0)),
        ],
        out_specs=pl.BlockSpec((_R, C), lambda i: (i, 0)),
        out_shape=jax.ShapeDtypeStruct((NP, C), jnp.float32),
    )(acc2p, degp, q2, W2, b2r)


def kernel(x, edge_index, edge_weight, W1, b1, W2, b2):
    src_p = jnp.pad(edge_index[0], (0, EP - E))
    dst_p = jnp.pad(edge_index[1], (0, EP - E))
    ew_p = jnp.pad(edge_weight, (0, EP - E))
    src3 = src_p.reshape(NW, NCH, CHE)
    dst3 = dst_p.reshape(NW, NCH, CHE)
    ew3 = ew_p.reshape(NW, NCH, CHE)
    dst2 = dst_p.reshape(NW, EPW)
    ew2 = ew_p.reshape(NW, EPW)
    x_pad = jnp.pad(x, ((0, NP - N), (0, 0)))

    degp = _sc_degree(dst2, ew2)
    g1 = _tc_prep(degp, x_pad, W1)
    acc1p = _sc_aggregate_h(g1, src3, dst3, ew3)
    q2 = _tc_mid(acc1p, degp, g1, b1.reshape(1, H))
    acc2p = _sc_aggregate_h(q2, src3, dst3, ew3)
    out = _tc_final(acc2p, degp, q2, W2, b2.reshape(1, C))
    return out[:N]


# speedup vs baseline: 1.8955x; 1.8955x over previous
import functools

import jax
import jax.numpy as jnp
from jax import lax
from jax.experimental import pallas as pl
from jax.experimental.pallas import tpu as pltpu
from jax.experimental.pallas import tpu_sc as plsc

N = 10000
NP = 10240
E = 320000
F_IN = 128
H = 128
C = 64

NC = 2
NS = 16
NW = NC * NS
L = 16
EPW = E // NW
B = 80
NBLK = EPW // B
RPT = NP // NS
CHB = 25
CHE = CHB * B
NCH = NBLK // CHB

_MESH = dict(core_axis_name="c", subcore_axis_name="s",
             num_cores=NC, num_subcores=NS)
_SC_PARAMS = pltpu.CompilerParams(needs_layout_passes=False)

_GDN = lax.GatherDimensionNumbers(
    offset_dims=(), collapsed_slice_dims=(0,), start_index_map=(0,))


def _lane_bcast(v16, j):
    return lax.gather(v16, jnp.full((L, 1), j, jnp.int32), _GDN, (1,),
                      mode=lax.GatherScatterMode.PROMISE_IN_BOUNDS)


def _sc_degree(dst2, ew2):
    @functools.partial(
        pl.kernel,
        out_type=jax.ShapeDtypeStruct((NW, NP), jnp.float32),
        mesh=plsc.VectorSubcoreMesh(**_MESH),
        compiler_params=_SC_PARAMS,
        scratch_types=[
            pltpu.VMEM((EPW,), jnp.int32),
            pltpu.VMEM((EPW,), jnp.float32),
            pltpu.VMEM((NP,), jnp.float32),
        ],
    )
    def k(dst_hbm, ew_hbm, out_hbm, dst_v, ew_v, acc):
        cid = lax.axis_index("c")
        sid = lax.axis_index("s")
        wid = sid * NC + cid
        pltpu.sync_copy(dst_hbm.at[wid], dst_v)
        pltpu.sync_copy(ew_hbm.at[wid], ew_v)

        zeros = jnp.zeros((L,), jnp.float32)

        def zbody(i, _):
            acc[pl.ds(i * L, L)] = zeros
            return 0

        lax.fori_loop(0, NP // L, zbody, 0)

        def ebody(i, _):
            idx = dst_v[pl.ds(i * L, L)]
            w = ew_v[pl.ds(i * L, L)]
            plsc.addupdate_scatter(acc, [idx], w)
            return 0

        lax.fori_loop(0, EPW // L, ebody, 0)
        pltpu.sync_copy(acc, out_hbm.at[wid])

    return k(dst2, ew2)


def _make_sc_aggregate(D):
    @functools.partial(
        pl.kernel,
        out_type=jax.ShapeDtypeStruct((NC, NP, D), jnp.float32),
        mesh=plsc.VectorSubcoreMesh(**_MESH),
        compiler_params=_SC_PARAMS,
        scratch_types=[
            pltpu.VMEM((CHE,), jnp.int32),
            pltpu.VMEM((CHE,), jnp.int32),
            pltpu.VMEM((CHE,), jnp.float32),
            pltpu.VMEM((B,), jnp.int32),
            pltpu.VMEM((B,), jnp.int32),
            pltpu.VMEM((B, D), jnp.float32),
            pltpu.VMEM_SHARED((NP, D), jnp.float32),
            pltpu.SemaphoreType.DMA,
        ],
    )
    def k(g_hbm, src_hbm, dst_hbm, ew_hbm, out_hbm,
          src_all, dst_all, ew_all, src_v, dst_v, rows, acc_sh, sem):
        cid = lax.axis_index("c")
        sid = lax.axis_index("s")
        wid = sid * NC + cid

        zeros = jnp.zeros((L,), jnp.float32)

        def zbody(i, _):
            for kk in range(D // L):
                rows[i, pl.ds(kk * L, L)] = zeros
            return 0

        lax.fori_loop(0, B, zbody, 0)
        rbase = sid * RPT
        for cchunk in range(RPT // B):
            pltpu.sync_copy(rows, acc_sh.at[pl.ds(rbase + cchunk * B, B)])
        plsc.subcore_barrier()

        def chunk(ch, _):
            pltpu.sync_copy(src_hbm.at[wid, ch], src_all)
            pltpu.sync_copy(dst_hbm.at[wid, ch], dst_all)
            pltpu.sync_copy(ew_hbm.at[wid, ch], ew_all)

            def block(b, _):
                off = b * B
                for g in range(B // L):
                    src_v[pl.ds(g * L, L)] = src_all[pl.ds(off + g * L, L)]
                    dst_v[pl.ds(g * L, L)] = dst_all[pl.ds(off + g * L, L)]
                pltpu.async_copy(g_hbm.at[src_v], rows, sem).wait()

                def sgroup(g, _):
                    w16 = ew_all[pl.ds(off + g * L, L)]
                    for j in range(L):
                        e = g * L + j
                        wspl = _lane_bcast(w16, j)
                        for kk in range(D // L):
                            rows[e, pl.ds(kk * L, L)] = (
                                rows[e, pl.ds(kk * L, L)] * wspl)
                    return 0

                lax.fori_loop(0, B // L, sgroup, 0)
                pltpu.sync_copy(rows, acc_sh.at[dst_v], add=True)
                return 0

            lax.fori_loop(0, CHB, block, 0)
            return 0

        lax.fori_loop(0, NCH, chunk, 0)
        plsc.subcore_barrier()
        pltpu.sync_copy(acc_sh.at[pl.ds(rbase, RPT)],
                        out_hbm.at[cid, pl.ds(rbase, RPT)])

    return k


_sc_aggregate_h = _make_sc_aggregate(H)


_R = 1024
_G = NP // _R


def _tc_prep(degp, x_pad, W1):
    def body(deg_ref, x_ref, w_ref, g_ref):
        deg = jnp.sum(deg_ref[...], axis=0) + 1.0
        di = lax.rsqrt(deg)[:, None]
        h = jnp.dot(x_ref[...], w_ref[...],
                    preferred_element_type=jnp.float32)
        g_ref[...] = di * h

    return pl.pallas_call(
        body,
        grid=(_G,),
        in_specs=[
            pl.BlockSpec((NW, _R), lambda i: (0, i)),
            pl.BlockSpec((_R, F_IN), lambda i: (i, 0)),
            pl.BlockSpec((F_IN, H), lambda i: (0, 0)),
        ],
        out_specs=pl.BlockSpec((_R, H), lambda i: (i, 0)),
        out_shape=jax.ShapeDtypeStruct((NP, H), jnp.float32),
    )(degp, x_pad, W1)


def _tc_mid(acc1p, degp, g1, b1r):
    def body(acc_ref, deg_ref, g_ref, b_ref, q2_ref):
        deg = jnp.sum(deg_ref[...], axis=0) + 1.0
        di = lax.rsqrt(deg)[:, None]
        a = acc_ref[0] + acc_ref[1] + g_ref[...]
        out1 = jnp.maximum(di * a + b_ref[...], 0.0)
        q2_ref[...] = di * out1

    return pl.pallas_call(
        body,
        grid=(_G,),
        in_specs=[
            pl.BlockSpec((NC, _R, H), lambda i: (0, i, 0)),
            pl.BlockSpec((NW, _R), lambda i: (0, i)),
            pl.BlockSpec((_R, H), lambda i: (i, 0)),
            pl.BlockSpec((1, H), lambda i: (0, 0)),
        ],
        out_specs=pl.BlockSpec((_R, H), lambda i: (i, 0)),
        out_shape=jax.ShapeDtypeStruct((NP, H), jnp.float32),
    )(acc1p, degp, g1, b1r)


def _tc_final(acc2p, degp, q2, W2, b2r):
    def body(acc_ref, deg_ref, q_ref, w_ref, b_ref, o_ref):
        deg = jnp.sum(deg_ref[...], axis=0) + 1.0
        di = lax.rsqrt(deg)[:, None]
        a = di * (acc_ref[0] + acc_ref[1] + q_ref[...])
        o = jnp.dot(a, w_ref[...],
                    preferred_element_type=jnp.float32) + b_ref[...]
        nrm = jnp.sqrt(jnp.sum(o * o, axis=1, keepdims=True))
        o_ref[...] = o / jnp.maximum(nrm, 1e-12)

    return pl.pallas_call(
        body,
        grid=(_G,),
        in_specs=[
            pl.BlockSpec((NC, _R, H), lambda i: (0, i, 0)),
            pl.BlockSpec((NW, _R), lambda i: (0, i)),
            pl.BlockSpec((_R, H), lambda i: (i, 0)),
            pl.BlockSpec((H, C), lambda i: (0, 0)),
            pl.BlockSpec((1, C), lambda i: (0, 0)),
        ],
        out_specs=pl.BlockSpec((_R, C), lambda i: (i, 0)),
        out_shape=jax.ShapeDtypeStruct((NP, C), jnp.float32),
    )(acc2p, degp, q2, W2, b2r)


def kernel(x, edge_index, edge_weight, W1, b1, W2, b2):
    src3 = edge_index[0].reshape(NW, NCH, CHE)
    dst3 = edge_index[1].reshape(NW, NCH, CHE)
    ew3 = edge_weight.reshape(NW, NCH, CHE)
    dst2 = edge_index[1].reshape(NW, EPW)
    ew2 = edge_weight.reshape(NW, EPW)
    x_pad = jnp.pad(x, ((0, NP - N), (0, 0)))

    degp = _sc_degree(dst2, ew2)
    g1 = _tc_prep(degp, x_pad, W1)
    acc1p = _sc_aggregate_h(g1, src3, dst3, ew3)
    q2 = _tc_mid(acc1p, degp, g1, b1.reshape(1, H))
    acc2p = _sc_aggregate_h(q2, src3, dst3, ew3)
    out = _tc_final(acc2p, degp, q2, W2, b2.reshape(1, C))
    return out[:N]
